# baseline (speedup 1.0000x reference)
import functools

import jax
import jax.numpy as jnp
import numpy as np
from jax import lax
from jax.experimental import pallas as pl
from jax.experimental.pallas import tpu as pltpu
from jax.experimental.pallas import tpu_sc as plsc

NBINS = 10
NGROUPS = 4
N = 8388608

NC = 2
NS = 16
L = 16
NW = NC * NS
SLAB = 16384
NSLAB = 9
PER_W = SLAB * NSLAB
N_SC = NW * PER_W
N_TC = N - N_SC
TC_COLS = 1024
TC_BR = 8
TC_ROW0 = N_SC // TC_COLS
NBUCK = 12
HISTW = NGROUPS * NBUCK * L

SEGS = 512
EDGE = 8.0
_xe = -EDGE + (2 * EDGE / SEGS) * np.arange(SEGS + 1)
_ue = 10.0 / (1.0 + np.exp(-_xe)) + 0.999
_A_TAB = ((_ue[1:] - _ue[:-1]) / (2 * EDGE / SEGS)).astype(np.float32)
_B_TAB = (_ue[:-1] - _A_TAB.astype(np.float64) * _xe[:-1]).astype(np.float32)

_mesh = plsc.VectorSubcoreMesh(core_axis_name="c", subcore_axis_name="s")


@functools.partial(
    pl.kernel,
    out_type=jax.ShapeDtypeStruct((NW, HISTW), jnp.float32),
    mesh=_mesh,
    scratch_types=[
        pltpu.VMEM((SLAB,), jnp.float32),
        pltpu.VMEM((SLAB,), jnp.float32),
        pltpu.VMEM((SLAB,), jnp.int32),
        pltpu.VMEM((SLAB,), jnp.int32),
        pltpu.VMEM((SEGS,), jnp.float32),
        pltpu.VMEM((SEGS,), jnp.float32),
        pltpu.VMEM((HISTW,), jnp.float32),
        pltpu.SemaphoreType.DMA,
        pltpu.SemaphoreType.DMA,
    ],
    compiler_params=pltpu.CompilerParams(needs_layout_passes=False),
)
def _hist_kernel(acts_hbm, lbl_hbm, ta_hbm, tb_hbm, out_hbm,
                 abuf0, abuf1, lbuf0, lbuf1, ta_v, tb_v, hist_v,
                 sem0, sem1):
    wid = lax.axis_index("s") * NC + lax.axis_index("c")
    base = wid * PER_W
    lane = lax.iota(jnp.int32, L)
    abufs = (abuf0, abuf1)
    lbufs = (lbuf0, lbuf1)
    sems = (sem0, sem1)

    def issue(s, b):
        sl = pl.ds(base + s * SLAB, SLAB)
        return (pltpu.async_copy(acts_hbm.at[sl], abufs[b], sems[b]),
                pltpu.async_copy(lbl_hbm.at[sl], lbufs[b], sems[b]))

    cps = [issue(0, 0), None]
    pltpu.sync_copy(ta_hbm, ta_v)
    pltpu.sync_copy(tb_hbm, tb_v)
    zero16 = jnp.zeros((L,), jnp.float32)
    for b in range(HISTW // L):
        hist_v[pl.ds(b * L, L)] = zero16

    U = 8

    def make_body(abuf, lbuf):
        def body(i, _):
            xs = [abuf[pl.ds((i * U + k) * L, L)] for k in range(U)]
            lbls = [lbuf[pl.ds((i * U + k) * L, L)] for k in range(U)]
            xc = [jnp.minimum(jnp.maximum(x, -8.0), 7.999) for x in xs]
            seg = [(x * (SEGS / (2 * EDGE)) + SEGS / 2).astype(jnp.int32)
                   for x in xc]
            a = [plsc.load_gather(ta_v, [s_]) for s_ in seg]
            bb = [plsc.load_gather(tb_v, [s_]) for s_ in seg]
            tp1 = [a[k] * xc[k] + bb[k] for k in range(U)]
            jp1 = [t.astype(jnp.int32) for t in tp1]
            frac = [tp1[k] - jp1[k].astype(jnp.float32) for k in range(U)]
            w1 = [0.1 * f for f in frac]
            w0 = [0.1 - w for w in w1]
            idx0 = [(lbls[k] * NBUCK + jp1[k]) * L + lane for k in range(U)]
            for k in range(U):
                plsc.addupdate_scatter(hist_v, [idx0[k]], w0[k])
                plsc.addupdate_scatter(hist_v, [idx0[k] + L], w1[k])
            return 0
        return body

    for s in range(NSLAB):
        b = s % 2
        if s + 1 < NSLAB:
            cps[1 - b] = issue(s + 1, 1 - b)
        for c in cps[b]:
            c.wait()
        lax.fori_loop(0, SLAB // (L * U), make_body(abufs[b], lbufs[b]), 0)

    pltpu.sync_copy(hist_v, out_hbm.at[wid])


def _tc_hist_body(x_ref, l_ref, o_ref, acc_ref):
    pid = pl.program_id(0)

    @pl.when(pid == 0)
    def _():
        acc_ref[...] = jnp.zeros_like(acc_ref)

    x = x_ref[...]
    lbl = l_ref[...]
    t = 10.0 / (1.0 + jnp.exp(-x)) - 0.001
    masks = [lbl == g for g in range(NGROUPS)]
    for i in range(NBINS):
        c = jnp.maximum(1.0 - jnp.abs(t - float(i)), 0.0)
        for g in range(NGROUPS):
            p = jnp.where(masks[g], c, 0.0)
            f = p[:, 0:128]
            for k in range(1, TC_COLS // 128):
                f = f + p[:, k * 128:(k + 1) * 128]
            acc_ref[i * NGROUPS + g] += f

    @pl.when(pid == pl.num_programs(0) - 1)
    def _():
        out = jnp.zeros((8, L), jnp.float32)
        rowi = lax.broadcasted_iota(jnp.int32, (8, L), 0)
        coli = lax.broadcasted_iota(jnp.int32, (8, L), 1)
        for g in range(NGROUPS):
            for i in range(NBINS):
                s = 0.1 * jnp.sum(acc_ref[i * NGROUPS + g])
                out = out + jnp.where((rowi == g) & (coli == i), s, 0.0)
        o_ref[...] = out


def _tc_hist(acts2, lbl2):
    return pl.pallas_call(
        _tc_hist_body,
        grid=(N_TC // (TC_BR * TC_COLS),),
        in_specs=[
            pl.BlockSpec((TC_BR, TC_COLS),
                         lambda i: (TC_ROW0 // TC_BR + i, 0)),
            pl.BlockSpec((TC_BR, TC_COLS),
                         lambda i: (TC_ROW0 // TC_BR + i, 0)),
        ],
        out_specs=pl.BlockSpec((8, L), lambda i: (0, 0)),
        out_shape=jax.ShapeDtypeStruct((8, L), jnp.float32),
        scratch_shapes=[
            pltpu.VMEM((NGROUPS * NBINS, TC_BR, 128), jnp.float32)],
    )(acts2, lbl2)


@functools.partial(
    pl.kernel,
    out_type=jax.ShapeDtypeStruct((L,), jnp.float32),
    mesh=_mesh,
    scratch_types=[
        pltpu.VMEM((NW * HISTW,), jnp.float32),
        pltpu.VMEM((L,), jnp.float32),
        pltpu.VMEM((8 * L,), jnp.float32),
        pltpu.VMEM((L,), jnp.float32),
    ],
    compiler_params=pltpu.CompilerParams(needs_layout_passes=False),
)
def _loss_kernel(parts_hbm, bary_hbm, tc_hbm, out_hbm,
                 parts_v, bary_v, tc_v, out_v):
    wid = lax.axis_index("s") * NC + lax.axis_index("c")

    @pl.when(wid == 0)
    def _():
        pltpu.sync_copy(parts_hbm, parts_v)
        pltpu.sync_copy(bary_hbm, bary_v)
        pltpu.sync_copy(tc_hbm, tc_v)
        lane = lax.iota(jnp.int32, L)
        lanef = lane.astype(jnp.float32)
        in10 = lanef < 10.0
        in9 = lanef < 9.0
        bary = jnp.where(in10, bary_v[...], 0.0)
        cb = plsc.cumsum(bary)
        qb = jnp.max(cb)
        Bm = jnp.where(in9, cb, 3.0)

        def pair_sum(U, V, Q):
            tot = 0.0
            for j in range(NBINS - 1):
                vj = jnp.sum(jnp.where(lane == j, V, 0.0))
                tot = tot + jnp.sum(
                    jnp.maximum(Q - jnp.maximum(U, vj), 0.0))
            return tot

        loss = 0.0
        for g in range(NGROUPS):
            offs = [(g * NBUCK + 1 + i) * L for i in range(NBINS)]

            def wbody(w, accs, offs=offs):
                return tuple(
                    accs[i] + parts_v[pl.ds(w * HISTW + offs[i], L)]
                    for i in range(NBINS))

            accs = lax.fori_loop(
                0, NW, wbody,
                tuple(jnp.zeros((L,), jnp.float32) for _ in range(NBINS)))
            h = jnp.where(in10, tc_v[pl.ds(g * L, L)], 0.0)
            for i in range(NBINS):
                h = h + jnp.where(lane == i, jnp.sum(accs[i]), 0.0)
            c = h + jnp.where(in10, 1e-4, 0.0)
            c = c / jnp.sum(c)
            c = c / jnp.sum(c)
            ca = plsc.cumsum(c)
            qa = jnp.max(ca)
            Q = jnp.maximum(qa, qb)
            Am = jnp.where(in9, ca, 3.0)
            loss = (loss + pair_sum(Am, Am, Q) + pair_sum(Bm, Bm, Q)
                    - 2.0 * pair_sum(Am, Bm, Q))

        out_v[...] = jnp.zeros((L,), jnp.float32) + loss
        pltpu.sync_copy(out_v, out_hbm)


def kernel(acts, group_labels, bary_est):
    parts = _hist_kernel(acts, group_labels,
                         jnp.asarray(_A_TAB), jnp.asarray(_B_TAB))
    tc_hist = _tc_hist(acts.reshape(-1, TC_COLS),
                       group_labels.reshape(-1, TC_COLS))
    barypad = jnp.concatenate(
        [bary_est[:, 0], jnp.zeros((L - NBINS,), jnp.float32)])
    lossv = _loss_kernel(parts.reshape(-1), barypad, tc_hist.reshape(-1))
    return (lossv[:1], bary_est)


# speedup vs baseline: 1.0180x; 1.0180x over previous
import functools

import jax
import jax.numpy as jnp
import numpy as np
from jax import lax
from jax.experimental import pallas as pl
from jax.experimental.pallas import tpu as pltpu
from jax.experimental.pallas import tpu_sc as plsc

NBINS = 10
NGROUPS = 4
N = 8388608

NC = 2
NS = 16
L = 16
NW = NC * NS
SLAB = 16384
NSLAB = 9
PER_W = SLAB * NSLAB
N_SC = NW * PER_W
N_TC = N - N_SC
TC_COLS = 1024
TC_BR = 8
TC_ROW0 = N_SC // TC_COLS
NBUCK = 12
HISTW = NGROUPS * NBUCK * L

SEGS = 512
EDGE = 8.0
_xe = -EDGE + (2 * EDGE / SEGS) * np.arange(SEGS + 1)
_ue = 10.0 / (1.0 + np.exp(-_xe)) + 0.999
_A_TAB = ((_ue[1:] - _ue[:-1]) / (2 * EDGE / SEGS)).astype(np.float32)
_B_TAB = (_ue[:-1] - _A_TAB.astype(np.float64) * _xe[:-1]).astype(np.float32)

_mesh = plsc.VectorSubcoreMesh(core_axis_name="c", subcore_axis_name="s")


@functools.partial(
    pl.kernel,
    out_type=jax.ShapeDtypeStruct((NW, HISTW), jnp.float32),
    mesh=_mesh,
    scratch_types=[
        pltpu.VMEM((SLAB,), jnp.float32),
        pltpu.VMEM((SLAB,), jnp.float32),
        pltpu.VMEM((SLAB,), jnp.int32),
        pltpu.VMEM((SLAB,), jnp.int32),
        pltpu.VMEM((SEGS,), jnp.float32),
        pltpu.VMEM((SEGS,), jnp.float32),
        pltpu.VMEM((HISTW,), jnp.float32),
        pltpu.SemaphoreType.DMA,
        pltpu.SemaphoreType.DMA,
    ],
    compiler_params=pltpu.CompilerParams(needs_layout_passes=False),
)
def _hist_kernel(acts_hbm, lbl_hbm, ta_hbm, tb_hbm, out_hbm,
                 abuf0, abuf1, lbuf0, lbuf1, ta_v, tb_v, hist_v,
                 sem0, sem1):
    wid = lax.axis_index("s") * NC + lax.axis_index("c")
    base = wid * PER_W
    lane = lax.iota(jnp.int32, L)
    abufs = (abuf0, abuf1)
    lbufs = (lbuf0, lbuf1)
    sems = (sem0, sem1)

    def issue(s, b):
        sl = pl.ds(base + s * SLAB, SLAB)
        return (pltpu.async_copy(acts_hbm.at[sl], abufs[b], sems[b]),
                pltpu.async_copy(lbl_hbm.at[sl], lbufs[b], sems[b]))

    cps = [issue(0, 0), None]
    pltpu.sync_copy(ta_hbm, ta_v)
    pltpu.sync_copy(tb_hbm, tb_v)
    zero16 = jnp.zeros((L,), jnp.float32)
    for b in range(HISTW // L):
        hist_v[pl.ds(b * L, L)] = zero16

    U = 8

    def make_body(abuf, lbuf):
        def body(i, _):
            xs = [abuf[pl.ds((i * U + k) * L, L)] for k in range(U)]
            lbls = [lbuf[pl.ds((i * U + k) * L, L)] for k in range(U)]
            xc = [jnp.minimum(jnp.maximum(x, -8.0), 7.999) for x in xs]
            seg = [(x * (SEGS / (2 * EDGE)) + SEGS / 2).astype(jnp.int32)
                   for x in xc]
            a = [plsc.load_gather(ta_v, [s_]) for s_ in seg]
            bb = [plsc.load_gather(tb_v, [s_]) for s_ in seg]
            tp1 = [a[k] * xc[k] + bb[k] for k in range(U)]
            jp1 = [t.astype(jnp.int32) for t in tp1]
            frac = [tp1[k] - jp1[k].astype(jnp.float32) for k in range(U)]
            w1 = [0.1 * f for f in frac]
            w0 = [0.1 - w for w in w1]
            idx0 = [(lbls[k] * NBUCK + jp1[k]) * L + lane for k in range(U)]
            for k in range(U):
                plsc.addupdate_scatter(hist_v, [idx0[k]], w0[k])
                plsc.addupdate_scatter(hist_v, [idx0[k] + L], w1[k])
            return 0
        return body

    for s in range(NSLAB):
        b = s % 2
        if s + 1 < NSLAB:
            cps[1 - b] = issue(s + 1, 1 - b)
        for c in cps[b]:
            c.wait()
        lax.fori_loop(0, SLAB // (L * U), make_body(abufs[b], lbufs[b]), 0)

    pltpu.sync_copy(hist_v, out_hbm.at[wid])


def _tc_hist_body(x_ref, l_ref, o_ref, acc_ref):
    pid = pl.program_id(0)

    @pl.when(pid == 0)
    def _():
        acc_ref[...] = jnp.zeros_like(acc_ref)

    x = x_ref[...]
    lbl = l_ref[...]
    t = 10.0 / (1.0 + jnp.exp(-x)) - 0.001
    mf = [(lbl == g).astype(jnp.float32) for g in range(NGROUPS)]
    for i in range(NBINS):
        c = jnp.maximum(1.0 - jnp.abs(t - float(i)), 0.0)
        for g in range(NGROUPS):
            p = c * mf[g]
            h1 = p[:, 0:512] + p[:, 512:1024]
            f = h1[:, 0:256] + h1[:, 256:512]
            acc_ref[i * NGROUPS + g] += f

    @pl.when(pid == pl.num_programs(0) - 1)
    def _():
        out = jnp.zeros((8, L), jnp.float32)
        rowi = lax.broadcasted_iota(jnp.int32, (8, L), 0)
        coli = lax.broadcasted_iota(jnp.int32, (8, L), 1)
        for g in range(NGROUPS):
            for i in range(NBINS):
                s = 0.1 * jnp.sum(acc_ref[i * NGROUPS + g])
                out = out + jnp.where((rowi == g) & (coli == i), s, 0.0)
        o_ref[...] = out


def _tc_hist(acts2, lbl2):
    return pl.pallas_call(
        _tc_hist_body,
        grid=(N_TC // (TC_BR * TC_COLS),),
        in_specs=[
            pl.BlockSpec((TC_BR, TC_COLS),
                         lambda i: (TC_ROW0 // TC_BR + i, 0)),
            pl.BlockSpec((TC_BR, TC_COLS),
                         lambda i: (TC_ROW0 // TC_BR + i, 0)),
        ],
        out_specs=pl.BlockSpec((8, L), lambda i: (0, 0)),
        out_shape=jax.ShapeDtypeStruct((8, L), jnp.float32),
        scratch_shapes=[
            pltpu.VMEM((NGROUPS * NBINS, TC_BR, TC_COLS // 4), jnp.float32)],
    )(acts2, lbl2)


@functools.partial(
    pl.kernel,
    out_type=jax.ShapeDtypeStruct((L,), jnp.float32),
    mesh=_mesh,
    scratch_types=[
        pltpu.VMEM((NW * HISTW,), jnp.float32),
        pltpu.VMEM((L,), jnp.float32),
        pltpu.VMEM((8 * L,), jnp.float32),
        pltpu.VMEM((L,), jnp.float32),
    ],
    compiler_params=pltpu.CompilerParams(needs_layout_passes=False),
)
def _loss_kernel(parts_hbm, bary_hbm, tc_hbm, out_hbm,
                 parts_v, bary_v, tc_v, out_v):
    wid = lax.axis_index("s") * NC + lax.axis_index("c")

    @pl.when(wid == 0)
    def _():
        pltpu.sync_copy(parts_hbm, parts_v)
        pltpu.sync_copy(bary_hbm, bary_v)
        pltpu.sync_copy(tc_hbm, tc_v)
        lane = lax.iota(jnp.int32, L)
        lanef = lane.astype(jnp.float32)
        in10 = lanef < 10.0
        in9 = lanef < 9.0
        bary = jnp.where(in10, bary_v[...], 0.0)
        cb = plsc.cumsum(bary)
        qb = jnp.max(cb)
        Bm = jnp.where(in9, cb, 3.0)

        def pair_sum(U, V, Q):
            tot = 0.0
            for j in range(NBINS - 1):
                vj = jnp.sum(jnp.where(lane == j, V, 0.0))
                tot = tot + jnp.sum(
                    jnp.maximum(Q - jnp.maximum(U, vj), 0.0))
            return tot

        loss = 0.0
        for g in range(NGROUPS):
            offs = [(g * NBUCK + 1 + i) * L for i in range(NBINS)]

            def wbody(w, accs, offs=offs):
                return tuple(
                    accs[i] + parts_v[pl.ds(w * HISTW + offs[i], L)]
                    for i in range(NBINS))

            accs = lax.fori_loop(
                0, NW, wbody,
                tuple(jnp.zeros((L,), jnp.float32) for _ in range(NBINS)))
            h = jnp.where(in10, tc_v[pl.ds(g * L, L)], 0.0)
            for i in range(NBINS):
                h = h + jnp.where(lane == i, jnp.sum(accs[i]), 0.0)
            c = h + jnp.where(in10, 1e-4, 0.0)
            c = c / jnp.sum(c)
            c = c / jnp.sum(c)
            ca = plsc.cumsum(c)
            qa = jnp.max(ca)
            Q = jnp.maximum(qa, qb)
            Am = jnp.where(in9, ca, 3.0)
            loss = (loss + pair_sum(Am, Am, Q) + pair_sum(Bm, Bm, Q)
                    - 2.0 * pair_sum(Am, Bm, Q))

        out_v[...] = jnp.zeros((L,), jnp.float32) + loss
        pltpu.sync_copy(out_v, out_hbm)


def kernel(acts, group_labels, bary_est):
    parts = _hist_kernel(acts, group_labels,
                         jnp.asarray(_A_TAB), jnp.asarray(_B_TAB))
    tc_hist = _tc_hist(acts.reshape(-1, TC_COLS),
                       group_labels.reshape(-1, TC_COLS))
    barypad = jnp.concatenate(
        [bary_est[:, 0], jnp.zeros((L - NBINS,), jnp.float32)])
    lossv = _loss_kernel(parts.reshape(-1), barypad, tc_hist.reshape(-1))
    return (lossv[:1], bary_est)


# speedup vs baseline: 2.4898x; 2.4459x over previous
import functools

import jax
import jax.numpy as jnp
import numpy as np
from jax import lax
from jax.experimental import pallas as pl
from jax.experimental.pallas import tpu as pltpu
from jax.experimental.pallas import tpu_sc as plsc

NBINS = 10
NGROUPS = 4
N = 8388608

NC = 2
NS = 16
L = 16
NW = NC * NS
SLAB = 16384
NSLAB = 16
PER_W = SLAB * NSLAB
NBUCK = 12
HISTW = NGROUPS * NBUCK * L

SEGS = 512
EDGE = 8.0
_xe = -EDGE + (2 * EDGE / SEGS) * np.arange(SEGS + 1)
_ue = 10.0 / (1.0 + np.exp(-_xe)) + 0.999
_A_TAB = ((_ue[1:] - _ue[:-1]) / (2 * EDGE / SEGS)).astype(np.float32)
_B_TAB = (_ue[:-1] - _A_TAB.astype(np.float64) * _xe[:-1]).astype(np.float32)

_mesh = plsc.VectorSubcoreMesh(core_axis_name="c", subcore_axis_name="s")


@functools.partial(
    pl.kernel,
    out_type=jax.ShapeDtypeStruct((NW, HISTW), jnp.float32),
    mesh=_mesh,
    scratch_types=[
        pltpu.VMEM((SLAB,), jnp.float32),
        pltpu.VMEM((SLAB,), jnp.float32),
        pltpu.VMEM((SLAB,), jnp.int32),
        pltpu.VMEM((SLAB,), jnp.int32),
        pltpu.VMEM((SEGS,), jnp.float32),
        pltpu.VMEM((SEGS,), jnp.float32),
        pltpu.VMEM((HISTW,), jnp.float32),
        pltpu.SemaphoreType.DMA,
        pltpu.SemaphoreType.DMA,
    ],
    compiler_params=pltpu.CompilerParams(needs_layout_passes=False),
)
def _hist_kernel(acts_hbm, lbl_hbm, ta_hbm, tb_hbm, out_hbm,
                 abuf0, abuf1, lbuf0, lbuf1, ta_v, tb_v, hist_v,
                 sem0, sem1):
    wid = lax.axis_index("s") * NC + lax.axis_index("c")
    base = wid * PER_W
    lane = lax.iota(jnp.int32, L)
    abufs = (abuf0, abuf1)
    lbufs = (lbuf0, lbuf1)
    sems = (sem0, sem1)

    def issue(s, b):
        sl = pl.ds(base + s * SLAB, SLAB)
        return (pltpu.async_copy(acts_hbm.at[sl], abufs[b], sems[b]),
                pltpu.async_copy(lbl_hbm.at[sl], lbufs[b], sems[b]))

    cps = [issue(0, 0), None]
    pltpu.sync_copy(ta_hbm, ta_v)
    pltpu.sync_copy(tb_hbm, tb_v)
    zero16 = jnp.zeros((L,), jnp.float32)
    for b in range(HISTW // L):
        hist_v[pl.ds(b * L, L)] = zero16

    U = 8

    def make_body(abuf, lbuf):
        def body(i, _):
            xs = [abuf[pl.ds((i * U + k) * L, L)] for k in range(U)]
            lbls = [lbuf[pl.ds((i * U + k) * L, L)] for k in range(U)]
            xc = [jnp.minimum(jnp.maximum(x, -8.0), 7.999) for x in xs]
            seg = [(x * (SEGS / (2 * EDGE)) + SEGS / 2).astype(jnp.int32)
                   for x in xc]
            a = [plsc.load_gather(ta_v, [s_]) for s_ in seg]
            bb = [plsc.load_gather(tb_v, [s_]) for s_ in seg]
            tp1 = [a[k] * xc[k] + bb[k] for k in range(U)]
            jp1 = [t.astype(jnp.int32) for t in tp1]
            frac = [tp1[k] - jp1[k].astype(jnp.float32) for k in range(U)]
            w1 = [0.1 * f for f in frac]
            w0 = [0.1 - w for w in w1]
            idx0 = [(lbls[k] * NBUCK + jp1[k]) * L + lane for k in range(U)]
            for k in range(U):
                plsc.addupdate_scatter(hist_v, [idx0[k]], w0[k])
                plsc.addupdate_scatter(hist_v, [idx0[k] + L], w1[k])
            return 0
        return body

    for s in range(NSLAB):
        b = s % 2
        if s + 1 < NSLAB:
            cps[1 - b] = issue(s + 1, 1 - b)
        for c in cps[b]:
            c.wait()
        lax.fori_loop(0, SLAB // (L * U), make_body(abufs[b], lbufs[b]), 0)

    pltpu.sync_copy(hist_v, out_hbm.at[wid])


@functools.partial(
    pl.kernel,
    out_type=jax.ShapeDtypeStruct((L,), jnp.float32),
    mesh=_mesh,
    scratch_types=[
        pltpu.VMEM((NW * HISTW,), jnp.float32),
        pltpu.VMEM((L,), jnp.float32),
        pltpu.VMEM((L,), jnp.float32),
    ],
    compiler_params=pltpu.CompilerParams(needs_layout_passes=False),
)
def _loss_kernel(parts_hbm, bary_hbm, out_hbm,
                 parts_v, bary_v, out_v):
    wid = lax.axis_index("s") * NC + lax.axis_index("c")

    @pl.when(wid == 0)
    def _():
        pltpu.sync_copy(parts_hbm, parts_v)
        pltpu.sync_copy(bary_hbm, bary_v)
        lane = lax.iota(jnp.int32, L)
        lanef = lane.astype(jnp.float32)
        in10 = lanef < 10.0
        in9 = lanef < 9.0
        bary = jnp.where(in10, bary_v[...], 0.0)
        cb = plsc.cumsum(bary)
        qb = jnp.max(cb)
        Bm = jnp.where(in9, cb, 3.0)

        def pair_sum(U, V, Q):
            tot = 0.0
            for j in range(NBINS - 1):
                vj = jnp.sum(jnp.where(lane == j, V, 0.0))
                tot = tot + jnp.sum(
                    jnp.maximum(Q - jnp.maximum(U, vj), 0.0))
            return tot

        loss = 0.0
        for g in range(NGROUPS):
            offs = [(g * NBUCK + 1 + i) * L for i in range(NBINS)]

            def wbody(w, accs, offs=offs):
                return tuple(
                    accs[i] + parts_v[pl.ds(w * HISTW + offs[i], L)]
                    for i in range(NBINS))

            accs = lax.fori_loop(
                0, NW, wbody,
                tuple(jnp.zeros((L,), jnp.float32) for _ in range(NBINS)))
            h = jnp.zeros((L,), jnp.float32)
            for i in range(NBINS):
                h = h + jnp.where(lane == i, jnp.sum(accs[i]), 0.0)
            c = h + jnp.where(in10, 1e-4, 0.0)
            c = c / jnp.sum(c)
            c = c / jnp.sum(c)
            ca = plsc.cumsum(c)
            qa = jnp.max(ca)
            Q = jnp.maximum(qa, qb)
            Am = jnp.where(in9, ca, 3.0)
            loss = (loss + pair_sum(Am, Am, Q) + pair_sum(Bm, Bm, Q)
                    - 2.0 * pair_sum(Am, Bm, Q))

        out_v[...] = jnp.zeros((L,), jnp.float32) + loss
        pltpu.sync_copy(out_v, out_hbm)


def kernel(acts, group_labels, bary_est):
    parts = _hist_kernel(acts, group_labels,
                         jnp.asarray(_A_TAB), jnp.asarray(_B_TAB))
    barypad = jnp.concatenate(
        [bary_est[:, 0], jnp.zeros((L - NBINS,), jnp.float32)])
    lossv = _loss_kernel(parts.reshape(-1), barypad)
    return (lossv[:1], bary_est)


# speedup vs baseline: 2.5092x; 1.0078x over previous
import functools

import jax
import jax.numpy as jnp
import numpy as np
from jax import lax
from jax.experimental import pallas as pl
from jax.experimental.pallas import tpu as pltpu
from jax.experimental.pallas import tpu_sc as plsc

NBINS = 10
NGROUPS = 4
N = 8388608

NC = 2
NS = 16
L = 16
NW = NC * NS
SLAB = 16384
NSLAB = 16
PER_W = SLAB * NSLAB
NBUCK = 12
HISTW = NGROUPS * NBUCK * L

SEGS = 512
EDGE = 8.0
_xe = -EDGE + (2 * EDGE / SEGS) * np.arange(SEGS + 1)
_ue = 10.0 / (1.0 + np.exp(-_xe)) + 0.999
_A_TAB = ((_ue[1:] - _ue[:-1]) / (2 * EDGE / SEGS)).astype(np.float32)
_B_TAB = (_ue[:-1] - _A_TAB.astype(np.float64) * _xe[:-1]).astype(np.float32)

_mesh = plsc.VectorSubcoreMesh(core_axis_name="c", subcore_axis_name="s")


@functools.partial(
    pl.kernel,
    out_type=jax.ShapeDtypeStruct((NW, HISTW), jnp.float32),
    mesh=_mesh,
    scratch_types=[
        pltpu.VMEM((SLAB,), jnp.float32),
        pltpu.VMEM((SLAB,), jnp.float32),
        pltpu.VMEM((SLAB,), jnp.int32),
        pltpu.VMEM((SLAB,), jnp.int32),
        pltpu.VMEM((SEGS,), jnp.float32),
        pltpu.VMEM((SEGS,), jnp.float32),
        pltpu.VMEM((HISTW,), jnp.float32),
        pltpu.SemaphoreType.DMA,
        pltpu.SemaphoreType.DMA,
    ],
    compiler_params=pltpu.CompilerParams(needs_layout_passes=False),
)
def _hist_kernel(acts_hbm, lbl_hbm, ta_hbm, tb_hbm, out_hbm,
                 abuf0, abuf1, lbuf0, lbuf1, ta_v, tb_v, hist_v,
                 sem0, sem1):
    wid = lax.axis_index("s") * NC + lax.axis_index("c")
    base = wid * PER_W
    lane = lax.iota(jnp.int32, L)
    abufs = (abuf0, abuf1)
    lbufs = (lbuf0, lbuf1)
    sems = (sem0, sem1)

    def issue(s, b):
        sl = pl.ds(base + s * SLAB, SLAB)
        return (pltpu.async_copy(acts_hbm.at[sl], abufs[b], sems[b]),
                pltpu.async_copy(lbl_hbm.at[sl], lbufs[b], sems[b]))

    cps = [issue(0, 0), None]
    pltpu.sync_copy(ta_hbm, ta_v)
    pltpu.sync_copy(tb_hbm, tb_v)
    zero16 = jnp.zeros((L,), jnp.float32)
    for b in range(HISTW // L):
        hist_v[pl.ds(b * L, L)] = zero16

    U = 16

    def make_body(abuf, lbuf):
        def body(i, _):
            xs = [abuf[pl.ds((i * U + k) * L, L)] for k in range(U)]
            lbls = [lbuf[pl.ds((i * U + k) * L, L)] for k in range(U)]
            xc = [jnp.minimum(jnp.maximum(x, -8.0), 7.999) for x in xs]
            seg = [(x * (SEGS / (2 * EDGE)) + SEGS / 2).astype(jnp.int32)
                   for x in xc]
            a = [plsc.load_gather(ta_v, [s_]) for s_ in seg]
            bb = [plsc.load_gather(tb_v, [s_]) for s_ in seg]
            tp1 = [a[k] * xc[k] + bb[k] for k in range(U)]
            jp1 = [t.astype(jnp.int32) for t in tp1]
            frac = [tp1[k] - jp1[k].astype(jnp.float32) for k in range(U)]
            w1 = [0.1 * f for f in frac]
            w0 = [0.1 - w for w in w1]
            idx0 = [(lbls[k] * NBUCK + jp1[k]) * L + lane for k in range(U)]
            for k in range(U):
                plsc.addupdate_scatter(hist_v, [idx0[k]], w0[k])
                plsc.addupdate_scatter(hist_v, [idx0[k] + L], w1[k])
            return 0
        return body

    for s in range(NSLAB):
        b = s % 2
        if s + 1 < NSLAB:
            cps[1 - b] = issue(s + 1, 1 - b)
        for c in cps[b]:
            c.wait()
        lax.fori_loop(0, SLAB // (L * U), make_body(abufs[b], lbufs[b]), 0)

    pltpu.sync_copy(hist_v, out_hbm.at[wid])


@functools.partial(
    pl.kernel,
    out_type=jax.ShapeDtypeStruct((L,), jnp.float32),
    mesh=_mesh,
    scratch_types=[
        pltpu.VMEM((NW * HISTW,), jnp.float32),
        pltpu.VMEM((L,), jnp.float32),
        pltpu.VMEM((L,), jnp.float32),
    ],
    compiler_params=pltpu.CompilerParams(needs_layout_passes=False),
)
def _loss_kernel(parts_hbm, bary_hbm, out_hbm,
                 parts_v, bary_v, out_v):
    wid = lax.axis_index("s") * NC + lax.axis_index("c")

    @pl.when(wid == 0)
    def _():
        pltpu.sync_copy(parts_hbm, parts_v)
        pltpu.sync_copy(bary_hbm, bary_v)
        lane = lax.iota(jnp.int32, L)
        lanef = lane.astype(jnp.float32)
        in10 = lanef < 10.0
        in9 = lanef < 9.0
        bary = jnp.where(in10, bary_v[...], 0.0)
        cb = plsc.cumsum(bary)
        qb = jnp.max(cb)
        Bm = jnp.where(in9, cb, 3.0)

        def pair_sum(U, V, Q):
            tot = 0.0
            for j in range(NBINS - 1):
                vj = jnp.sum(jnp.where(lane == j, V, 0.0))
                tot = tot + jnp.sum(
                    jnp.maximum(Q - jnp.maximum(U, vj), 0.0))
            return tot

        loss = 0.0
        for g in range(NGROUPS):
            offs = [(g * NBUCK + 1 + i) * L for i in range(NBINS)]

            def wbody(w, accs, offs=offs):
                return tuple(
                    accs[i] + parts_v[pl.ds(w * HISTW + offs[i], L)]
                    for i in range(NBINS))

            accs = lax.fori_loop(
                0, NW, wbody,
                tuple(jnp.zeros((L,), jnp.float32) for _ in range(NBINS)))
            h = jnp.zeros((L,), jnp.float32)
            for i in range(NBINS):
                h = h + jnp.where(lane == i, jnp.sum(accs[i]), 0.0)
            c = h + jnp.where(in10, 1e-4, 0.0)
            c = c / jnp.sum(c)
            c = c / jnp.sum(c)
            ca = plsc.cumsum(c)
            qa = jnp.max(ca)
            Q = jnp.maximum(qa, qb)
            Am = jnp.where(in9, ca, 3.0)
            loss = (loss + pair_sum(Am, Am, Q) + pair_sum(Bm, Bm, Q)
                    - 2.0 * pair_sum(Am, Bm, Q))

        out_v[...] = jnp.zeros((L,), jnp.float32) + loss
        pltpu.sync_copy(out_v, out_hbm)


def kernel(acts, group_labels, bary_est):
    parts = _hist_kernel(acts, group_labels,
                         jnp.asarray(_A_TAB), jnp.asarray(_B_TAB))
    barypad = jnp.concatenate(
        [bary_est[:, 0], jnp.zeros((L - NBINS,), jnp.float32)])
    lossv = _loss_kernel(parts.reshape(-1), barypad)
    return (lossv[:1], bary_est)


# speedup vs baseline: 2.5533x; 1.0176x over previous
import functools

import jax
import jax.numpy as jnp
import numpy as np
from jax import lax
from jax.experimental import pallas as pl
from jax.experimental.pallas import tpu as pltpu
from jax.experimental.pallas import tpu_sc as plsc

NBINS = 10
NGROUPS = 4
N = 8388608

NC = 2
NS = 16
L = 16
NW = NC * NS
SLAB = 16384
NSLAB = 16
PER_W = SLAB * NSLAB
NBUCK = 12
HISTW = NGROUPS * NBUCK * L

SEGS = 512
EDGE = 8.0
_xe = -EDGE + (2 * EDGE / SEGS) * np.arange(SEGS + 1)
_ue = 10.0 / (1.0 + np.exp(-_xe)) + 0.999
_A_TAB = ((_ue[1:] - _ue[:-1]) / (2 * EDGE / SEGS)).astype(np.float32)
_B_TAB = (_ue[:-1] - _A_TAB.astype(np.float64) * _xe[:-1]).astype(np.float32)

_mesh = plsc.VectorSubcoreMesh(core_axis_name="c", subcore_axis_name="s")


@functools.partial(
    pl.kernel,
    out_type=jax.ShapeDtypeStruct((NW, HISTW), jnp.float32),
    mesh=_mesh,
    scratch_types=[
        pltpu.VMEM((SLAB,), jnp.float32),
        pltpu.VMEM((SLAB,), jnp.float32),
        pltpu.VMEM((SLAB,), jnp.int32),
        pltpu.VMEM((SLAB,), jnp.int32),
        pltpu.VMEM((SEGS,), jnp.float32),
        pltpu.VMEM((SEGS,), jnp.float32),
        pltpu.VMEM((HISTW,), jnp.float32),
        pltpu.SemaphoreType.DMA,
        pltpu.SemaphoreType.DMA,
    ],
    compiler_params=pltpu.CompilerParams(needs_layout_passes=False),
)
def _hist_kernel(acts_hbm, lbl_hbm, ta_hbm, tb_hbm, out_hbm,
                 abuf0, abuf1, lbuf0, lbuf1, ta_v, tb_v, hist_v,
                 sem0, sem1):
    wid = lax.axis_index("s") * NC + lax.axis_index("c")
    base = wid * PER_W
    lane = lax.iota(jnp.int32, L)
    abufs = (abuf0, abuf1)
    lbufs = (lbuf0, lbuf1)
    sems = (sem0, sem1)

    def issue(s, b):
        sl = pl.ds(base + s * SLAB, SLAB)
        return (pltpu.async_copy(acts_hbm.at[sl], abufs[b], sems[b]),
                pltpu.async_copy(lbl_hbm.at[sl], lbufs[b], sems[b]))

    cps = [issue(0, 0), None]
    pltpu.sync_copy(ta_hbm, ta_v)
    pltpu.sync_copy(tb_hbm, tb_v)
    zero16 = jnp.zeros((L,), jnp.float32)
    for b in range(HISTW // L):
        hist_v[pl.ds(b * L, L)] = zero16

    U = 16

    def make_body(abuf, lbuf):
        def body(i, _):
            xs = [abuf[pl.ds((i * U + k) * L, L)] for k in range(U)]
            lbls = [lbuf[pl.ds((i * U + k) * L, L)] for k in range(U)]
            xc = [jnp.minimum(jnp.maximum(x, -8.0), 7.999) for x in xs]
            seg = [(x * (SEGS / (2 * EDGE)) + SEGS / 2).astype(jnp.int32)
                   for x in xc]
            a = [plsc.load_gather(ta_v, [s_]) for s_ in seg]
            bb = [plsc.load_gather(tb_v, [s_]) for s_ in seg]
            tp1 = [a[k] * xc[k] + bb[k] for k in range(U)]
            jp1 = [t.astype(jnp.int32) for t in tp1]
            frac = [tp1[k] - jp1[k].astype(jnp.float32) for k in range(U)]
            w1 = frac
            w0 = [1.0 - f for f in frac]
            idx0 = [(lbls[k] * NBUCK + jp1[k]) * L + lane for k in range(U)]
            for k in range(U):
                plsc.addupdate_scatter(hist_v, [idx0[k]], w0[k])
                plsc.addupdate_scatter(hist_v, [idx0[k] + L], w1[k])
            return 0
        return body

    for s in range(NSLAB):
        b = s % 2
        if s + 1 < NSLAB:
            cps[1 - b] = issue(s + 1, 1 - b)
        for c in cps[b]:
            c.wait()
        lax.fori_loop(0, SLAB // (L * U), make_body(abufs[b], lbufs[b]), 0)

    pltpu.sync_copy(hist_v, out_hbm.at[wid])


@functools.partial(
    pl.kernel,
    out_type=jax.ShapeDtypeStruct((L,), jnp.float32),
    mesh=_mesh,
    scratch_types=[
        pltpu.VMEM((NW * HISTW,), jnp.float32),
        pltpu.VMEM((L,), jnp.float32),
        pltpu.VMEM((L,), jnp.float32),
    ],
    compiler_params=pltpu.CompilerParams(needs_layout_passes=False),
)
def _loss_kernel(parts_hbm, bary_hbm, out_hbm,
                 parts_v, bary_v, out_v):
    wid = lax.axis_index("s") * NC + lax.axis_index("c")

    @pl.when(wid == 0)
    def _():
        pltpu.sync_copy(parts_hbm, parts_v)
        pltpu.sync_copy(bary_hbm, bary_v)
        lane = lax.iota(jnp.int32, L)
        lanef = lane.astype(jnp.float32)
        in10 = lanef < 10.0
        in9 = lanef < 9.0
        bary = jnp.where(in10, bary_v[...], 0.0)
        cb = plsc.cumsum(bary)
        qb = jnp.max(cb)
        Bm = jnp.where(in9, cb, 3.0)

        def pair_sum(U, V, Q):
            tot = 0.0
            for j in range(NBINS - 1):
                vj = jnp.sum(jnp.where(lane == j, V, 0.0))
                tot = tot + jnp.sum(
                    jnp.maximum(Q - jnp.maximum(U, vj), 0.0))
            return tot

        loss = 0.0
        for g in range(NGROUPS):
            offs = [(g * NBUCK + 1 + i) * L for i in range(NBINS)]

            def wbody(w, accs, offs=offs):
                return tuple(
                    accs[i] + parts_v[pl.ds(w * HISTW + offs[i], L)]
                    for i in range(NBINS))

            accs = lax.fori_loop(
                0, NW, wbody,
                tuple(jnp.zeros((L,), jnp.float32) for _ in range(NBINS)))
            h = jnp.zeros((L,), jnp.float32)
            for i in range(NBINS):
                h = h + jnp.where(lane == i, jnp.sum(accs[i]), 0.0)
            c = 0.1 * h + jnp.where(in10, 1e-4, 0.0)
            c = c / jnp.sum(c)
            c = c / jnp.sum(c)
            ca = plsc.cumsum(c)
            qa = jnp.max(ca)
            Q = jnp.maximum(qa, qb)
            Am = jnp.where(in9, ca, 3.0)
            loss = (loss + pair_sum(Am, Am, Q) + pair_sum(Bm, Bm, Q)
                    - 2.0 * pair_sum(Am, Bm, Q))

        out_v[...] = jnp.zeros((L,), jnp.float32) + loss
        pltpu.sync_copy(out_v, out_hbm)


def kernel(acts, group_labels, bary_est):
    parts = _hist_kernel(acts, group_labels,
                         jnp.asarray(_A_TAB), jnp.asarray(_B_TAB))
    barypad = jnp.concatenate(
        [bary_est[:, 0], jnp.zeros((L - NBINS,), jnp.float32)])
    lossv = _loss_kernel(parts.reshape(-1), barypad)
    return (lossv[:1], bary_est)
